# baseline (device time: 284846 ns/iter reference)
import functools

import jax
import jax.numpy as jnp
from jax import lax
from jax.experimental import pallas as pl
from jax.experimental.pallas import tpu as pltpu

N_DEV = 8
TC = 64


def kernel(x, A, B, C):
    b, s, d = x.shape
    n = B.shape[-1]

    dA_nd = jnp.exp(A.T)

    def body(x_ref, da_ref, b_ref, c_ref, out_ref, hend_ref, hin_ref,
             send_sem, recv_sem):
        my = lax.axis_index("i")
        left = lax.rem(my + N_DEV - 1, N_DEV)
        right = lax.rem(my + 1, N_DEV)

        barrier_sem = pltpu.get_barrier_semaphore()
        for nbr in (left, right):
            pl.semaphore_signal(
                barrier_sem, inc=1,
                device_id=(nbr,), device_id_type=pl.DeviceIdType.MESH,
            )
        pl.semaphore_wait(barrier_sem, 2)

        dA = da_ref[:, :]

        def step(t, h):
            x_t = x_ref[:, pl.ds(t, 1), :][:, 0, :]
            b_t = b_ref[:, pl.ds(t, 1), :][:, 0, :]
            c_t = c_ref[:, pl.ds(t, 1), :][:, 0, :]
            h = h * dA[None] + x_t[:, None, :] * b_t[:, :, None]
            y_t = jnp.sum(h * c_t[:, :, None], axis=1)
            out_ref[:, pl.ds(t, 1), :] = y_t[:, None, :]
            return h

        h0 = jnp.zeros((b, n, d), jnp.float32)
        h_end = lax.fori_loop(0, s, step, h0)
        hend_ref[:, :, :] = h_end

        rdma = pltpu.make_async_remote_copy(
            src_ref=hend_ref,
            dst_ref=hin_ref,
            send_sem=send_sem,
            recv_sem=recv_sem,
            device_id=(right,),
            device_id_type=pl.DeviceIdType.MESH,
        )
        rdma.start()
        rdma.wait()

        @pl.when(my != 0)
        def _():
            h_in = hin_ref[:, :, :]

            def cstep(t, g):
                c_t = c_ref[:, pl.ds(t, 1), :][:, 0, :]
                contrib = jnp.sum(
                    (g[None] * h_in) * c_t[:, :, None], axis=1
                )
                cur = out_ref[:, pl.ds(t, 1), :]
                out_ref[:, pl.ds(t, 1), :] = cur + contrib[:, None, :]
                return g * dA

            lax.fori_loop(0, TC, cstep, dA)

        @functools.partial(
            pl.run_scoped, sem2=pltpu.SemaphoreType.REGULAR
        )
        def _(sem2):
            for nbr in (left, right):
                pl.semaphore_signal(
                    sem2, inc=1,
                    device_id=(nbr,), device_id_type=pl.DeviceIdType.MESH,
                )
            pl.semaphore_wait(sem2, 2)

    return pl.pallas_call(
        body,
        out_shape=jax.ShapeDtypeStruct((b, s, d), jnp.float32),
        in_specs=[
            pl.BlockSpec(memory_space=pltpu.VMEM),
            pl.BlockSpec(memory_space=pltpu.VMEM),
            pl.BlockSpec(memory_space=pltpu.VMEM),
            pl.BlockSpec(memory_space=pltpu.VMEM),
        ],
        out_specs=pl.BlockSpec(memory_space=pltpu.VMEM),
        scratch_shapes=[
            pltpu.VMEM((b, n, d), jnp.float32),
            pltpu.VMEM((b, n, d), jnp.float32),
            pltpu.SemaphoreType.DMA,
            pltpu.SemaphoreType.DMA,
        ],
        compiler_params=pltpu.CompilerParams(collective_id=0),
    )(x, dA_nd, B, C)


# device time: 196521 ns/iter; 1.4494x vs baseline; 1.4494x over previous
import functools

import jax
import jax.numpy as jnp
from jax import lax
from jax.experimental import pallas as pl
from jax.experimental.pallas import tpu as pltpu

N_DEV = 8
TC = 64


def kernel(x, A, B, C):
    b, s, d = x.shape
    n = B.shape[-1]

    dA_nd = jnp.exp(A.T)

    def body(x_ref, da_ref, b_ref, c_ref, out_ref, hend_ref, hin_ref,
             send_sem, recv_sem):
        my = lax.axis_index("i")
        left = lax.rem(my + N_DEV - 1, N_DEV)
        right = lax.rem(my + 1, N_DEV)

        barrier_sem = pltpu.get_barrier_semaphore()
        for nbr in (left, right):
            pl.semaphore_signal(
                barrier_sem, inc=1,
                device_id=(nbr,), device_id_type=pl.DeviceIdType.MESH,
            )
        pl.semaphore_wait(barrier_sem, 2)

        dA = da_ref[:, :]

        def step(t, h):
            x_t = x_ref[:, pl.ds(t, 1), :][:, 0, :]
            b_t = b_ref[:, pl.ds(t, 1), :][:, 0, :]
            c_t = c_ref[:, pl.ds(t, 1), :][:, 0, :]
            h = h * dA[None] + x_t[:, None, :] * b_t[:, :, None]
            y_t = jnp.sum(h * c_t[:, :, None], axis=1)
            out_ref[:, pl.ds(t, 1), :] = y_t[:, None, :]
            return h

        h0 = jnp.zeros((b, n, d), jnp.float32)
        h_end = lax.fori_loop(0, s, step, h0, unroll=8)
        hend_ref[:, :, :] = h_end

        rdma = pltpu.make_async_remote_copy(
            src_ref=hend_ref,
            dst_ref=hin_ref,
            send_sem=send_sem,
            recv_sem=recv_sem,
            device_id=(right,),
            device_id_type=pl.DeviceIdType.MESH,
        )
        rdma.start()
        rdma.wait()

        @pl.when(my != 0)
        def _():
            h_in = hin_ref[:, :, :]

            def cstep(t, g):
                c_t = c_ref[:, pl.ds(t, 1), :][:, 0, :]
                contrib = jnp.sum(
                    (g[None] * h_in) * c_t[:, :, None], axis=1
                )
                cur = out_ref[:, pl.ds(t, 1), :]
                out_ref[:, pl.ds(t, 1), :] = cur + contrib[:, None, :]
                return g * dA

            lax.fori_loop(0, TC, cstep, dA)

        @functools.partial(
            pl.run_scoped, sem2=pltpu.SemaphoreType.REGULAR
        )
        def _(sem2):
            for nbr in (left, right):
                pl.semaphore_signal(
                    sem2, inc=1,
                    device_id=(nbr,), device_id_type=pl.DeviceIdType.MESH,
                )
            pl.semaphore_wait(sem2, 2)

    return pl.pallas_call(
        body,
        out_shape=jax.ShapeDtypeStruct((b, s, d), jnp.float32),
        in_specs=[
            pl.BlockSpec(memory_space=pltpu.VMEM),
            pl.BlockSpec(memory_space=pltpu.VMEM),
            pl.BlockSpec(memory_space=pltpu.VMEM),
            pl.BlockSpec(memory_space=pltpu.VMEM),
        ],
        out_specs=pl.BlockSpec(memory_space=pltpu.VMEM),
        scratch_shapes=[
            pltpu.VMEM((b, n, d), jnp.float32),
            pltpu.VMEM((b, n, d), jnp.float32),
            pltpu.SemaphoreType.DMA,
            pltpu.SemaphoreType.DMA,
        ],
        compiler_params=pltpu.CompilerParams(collective_id=0),
    )(x, dA_nd, B, C)


# device time: 166795 ns/iter; 1.7078x vs baseline; 1.1782x over previous
import functools

import jax
import jax.numpy as jnp
from jax import lax
from jax.experimental import pallas as pl
from jax.experimental.pallas import tpu as pltpu

N_DEV = 8
TC = 64


def kernel(x, A, B, C):
    b, s, d = x.shape
    n = B.shape[-1]

    dA_nd = jnp.exp(A.T)
    x = x.astype(jnp.bfloat16)
    B = B.astype(jnp.bfloat16)
    C = C.astype(jnp.bfloat16)

    def body(x_ref, da_ref, b_ref, c_ref, out_ref, hend_ref, hin_ref,
             send_sem, recv_sem):
        my = lax.axis_index("i")
        left = lax.rem(my + N_DEV - 1, N_DEV)
        right = lax.rem(my + 1, N_DEV)

        barrier_sem = pltpu.get_barrier_semaphore()
        for nbr in (left, right):
            pl.semaphore_signal(
                barrier_sem, inc=1,
                device_id=(nbr,), device_id_type=pl.DeviceIdType.MESH,
            )
        pl.semaphore_wait(barrier_sem, 2)

        dA = da_ref[:, :]
        dA_bf = dA.astype(jnp.bfloat16)

        TB = 8

        def step(tb, h):
            t0 = tb * TB
            x_blk = x_ref[:, pl.ds(t0, TB), :]
            b_blk = b_ref[:, pl.ds(t0, TB), :]
            c_blk = c_ref[:, pl.ds(t0, TB), :]
            ys = []
            for j in range(TB):
                h = (
                    h * dA_bf[None]
                    + x_blk[:, j, :][:, None, :] * b_blk[:, j, :][:, :, None]
                )
                ys.append(
                    jnp.sum(
                        h * c_blk[:, j, :][:, :, None],
                        axis=1,
                        dtype=jnp.float32,
                    )[:, None, :]
                )
            out_ref[:, pl.ds(t0, TB), :] = jnp.concatenate(ys, axis=1)
            return h

        h0 = jnp.zeros((b, n, d), jnp.bfloat16)
        h_end = lax.fori_loop(0, s // TB, step, h0)
        hend_ref[:, :, :] = h_end

        rdma = pltpu.make_async_remote_copy(
            src_ref=hend_ref,
            dst_ref=hin_ref,
            send_sem=send_sem,
            recv_sem=recv_sem,
            device_id=(right,),
            device_id_type=pl.DeviceIdType.MESH,
        )
        rdma.start()
        rdma.wait()

        @pl.when(my != 0)
        def _():
            h_in = hin_ref[:, :, :].astype(jnp.float32)

            def cstep(tb, g):
                t0 = tb * TB
                c_blk = c_ref[:, pl.ds(t0, TB), :].astype(
                    jnp.float32
                )
                cur = out_ref[:, pl.ds(t0, TB), :]
                adds = []
                for j in range(TB):
                    adds.append(
                        jnp.sum(
                            (g[None] * h_in) * c_blk[:, j, :][:, :, None],
                            axis=1,
                        )[:, None, :]
                    )
                    g = g * dA
                out_ref[:, pl.ds(t0, TB), :] = cur + jnp.concatenate(
                    adds, axis=1
                )
                return g

            lax.fori_loop(0, TC // TB, cstep, dA)

        @functools.partial(
            pl.run_scoped, sem2=pltpu.SemaphoreType.REGULAR
        )
        def _(sem2):
            for nbr in (left, right):
                pl.semaphore_signal(
                    sem2, inc=1,
                    device_id=(nbr,), device_id_type=pl.DeviceIdType.MESH,
                )
            pl.semaphore_wait(sem2, 2)

    return pl.pallas_call(
        body,
        out_shape=jax.ShapeDtypeStruct((b, s, d), jnp.float32),
        in_specs=[
            pl.BlockSpec(memory_space=pltpu.VMEM),
            pl.BlockSpec(memory_space=pltpu.VMEM),
            pl.BlockSpec(memory_space=pltpu.VMEM),
            pl.BlockSpec(memory_space=pltpu.VMEM),
        ],
        out_specs=pl.BlockSpec(memory_space=pltpu.VMEM),
        scratch_shapes=[
            pltpu.VMEM((b, n, d), jnp.bfloat16),
            pltpu.VMEM((b, n, d), jnp.bfloat16),
            pltpu.SemaphoreType.DMA,
            pltpu.SemaphoreType.DMA,
        ],
        compiler_params=pltpu.CompilerParams(collective_id=0),
    )(x, dA_nd, B, C)


# device time: 145356 ns/iter; 1.9596x vs baseline; 1.1475x over previous
import functools

import jax
import jax.numpy as jnp
from jax import lax
from jax.experimental import pallas as pl
from jax.experimental.pallas import tpu as pltpu

N_DEV = 8
TC = 64


def kernel(x, A, B, C):
    b, s, d = x.shape
    n = B.shape[-1]

    dA_nd = jnp.exp(A.T)

    def body(x_ref, da_ref, b_ref, c_ref, out_ref, hend_ref, hin_ref,
             send_sem, recv_sem):
        my = lax.axis_index("i")
        left = lax.rem(my + N_DEV - 1, N_DEV)
        right = lax.rem(my + 1, N_DEV)

        barrier_sem = pltpu.get_barrier_semaphore()
        for nbr in (left, right):
            pl.semaphore_signal(
                barrier_sem, inc=1,
                device_id=(nbr,), device_id_type=pl.DeviceIdType.MESH,
            )
        pl.semaphore_wait(barrier_sem, 2)

        dA = da_ref[:, :]
        dA_bf = dA.astype(jnp.bfloat16)

        TB = 16

        def step(tb, h):
            t0 = tb * TB
            x_blk = x_ref[:, pl.ds(t0, TB), :].astype(
                jnp.bfloat16
            )
            b_blk = b_ref[:, pl.ds(t0, TB), :].astype(
                jnp.bfloat16
            )
            c_blk = c_ref[:, pl.ds(t0, TB), :].astype(
                jnp.bfloat16
            )
            ys = []
            for j in range(TB):
                h = (
                    h * dA_bf[None]
                    + x_blk[:, j, :][:, None, :] * b_blk[:, j, :][:, :, None]
                )
                ys.append(
                    jnp.sum(
                        h * c_blk[:, j, :][:, :, None],
                        axis=1,
                        dtype=jnp.float32,
                    )[:, None, :]
                )
            out_ref[:, pl.ds(t0, TB), :] = jnp.concatenate(ys, axis=1)
            return h

        h0 = jnp.zeros((b, n, d), jnp.bfloat16)
        h_end = lax.fori_loop(0, s // TB, step, h0)
        hend_ref[:, :, :] = h_end

        rdma = pltpu.make_async_remote_copy(
            src_ref=hend_ref,
            dst_ref=hin_ref,
            send_sem=send_sem,
            recv_sem=recv_sem,
            device_id=(right,),
            device_id_type=pl.DeviceIdType.MESH,
        )
        rdma.start()
        rdma.wait()

        @pl.when(my != 0)
        def _():
            h_in = hin_ref[:, :, :].astype(jnp.float32)

            def cstep(tb, g):
                t0 = tb * TB
                c_blk = c_ref[:, pl.ds(t0, TB), :].astype(
                    jnp.float32
                )
                cur = out_ref[:, pl.ds(t0, TB), :]
                adds = []
                for j in range(TB):
                    adds.append(
                        jnp.sum(
                            (g[None] * h_in) * c_blk[:, j, :][:, :, None],
                            axis=1,
                        )[:, None, :]
                    )
                    g = g * dA
                out_ref[:, pl.ds(t0, TB), :] = cur + jnp.concatenate(
                    adds, axis=1
                )
                return g

            lax.fori_loop(0, TC // TB, cstep, dA)

        @functools.partial(
            pl.run_scoped, sem2=pltpu.SemaphoreType.REGULAR
        )
        def _(sem2):
            for nbr in (left, right):
                pl.semaphore_signal(
                    sem2, inc=1,
                    device_id=(nbr,), device_id_type=pl.DeviceIdType.MESH,
                )
            pl.semaphore_wait(sem2, 2)

    return pl.pallas_call(
        body,
        out_shape=jax.ShapeDtypeStruct((b, s, d), jnp.float32),
        in_specs=[
            pl.BlockSpec(memory_space=pltpu.VMEM),
            pl.BlockSpec(memory_space=pltpu.VMEM),
            pl.BlockSpec(memory_space=pltpu.VMEM),
            pl.BlockSpec(memory_space=pltpu.VMEM),
        ],
        out_specs=pl.BlockSpec(memory_space=pltpu.VMEM),
        scratch_shapes=[
            pltpu.VMEM((b, n, d), jnp.bfloat16),
            pltpu.VMEM((b, n, d), jnp.bfloat16),
            pltpu.SemaphoreType.DMA,
            pltpu.SemaphoreType.DMA,
        ],
        compiler_params=pltpu.CompilerParams(collective_id=0),
    )(x, dA_nd, B, C)
